# Initial kernel scaffold; baseline (speedup 1.0000x reference)
import jax
import jax.numpy as jnp
from jax.experimental import pallas as pl


def kernel(values, row_indices, col_indices, W):
    raise NotImplementedError("write your pallas kernel here")


# speedup vs baseline: 14.1920x; 14.1920x over previous
import functools

import jax
import jax.numpy as jnp
from jax import lax
from jax.experimental import pallas as pl
from jax.experimental.pallas import tpu as pltpu
from jax.experimental.pallas import tpu_sc as plsc

BATCH = 65536
INP = 65536
OUT = 64
NNZ = 4194304

NW = 32
VW_PER_W = 8
NVW = NW * VW_PER_W
ROWS_PER_VW = BATCH // NVW
B = 512
GCH = 128


def _sc_body(val_hbm, row_hbm, col_hbm, w_hbm, off_hbm, out_hbm,
             colv, g, win, rows_v, vals_v, off_v, sem):
    wid = lax.axis_index("s") * 2 + lax.axis_index("c")

    pltpu.sync_copy(off_hbm.at[pl.ds(wid * VW_PER_W, 16)], off_v)
    ov = off_v[0:16]

    zvec = jnp.zeros((16,), jnp.float32)

    for i in range(VW_PER_W):
        row_base = (wid * VW_PER_W + i) * ROWS_PER_VW
        lo = ov[i]
        hi = ov[i + 1]

        def zero_body(zi, _):
            win[zi, 0:16] = zvec
            win[zi, 16:32] = zvec
            win[zi, 32:48] = zvec
            win[zi, 48:64] = zvec
            return 0
        lax.fori_loop(0, ROWS_PER_VW, zero_body, 0)

        s0 = (lo // 8) * 8
        nblk = (hi - s0 + B - 1) // B

        def blk_body(b, carry):
            a0, a1, a2, a3, prev = carry
            start_u = s0 + b * B
            start = jnp.minimum(start_u, NNZ - B)
            start = pl.multiple_of(start, 8)
            pltpu.sync_copy(col_hbm.at[pl.ds(start, B)], colv)
            pltpu.sync_copy(row_hbm.at[pl.ds(start, B)], rows_v)
            pltpu.sync_copy(val_hbm.at[pl.ds(start, B)], vals_v)
            descs = []
            for c in range(B // GCH):
                descs.append(pltpu.async_copy(
                    w_hbm.at[colv.at[pl.ds(c * GCH, GCH)]],
                    g.at[pl.ds(c * GCH, GCH)], sem))
            for d in descs:
                d.wait()

            lo_j = jnp.maximum(lo, start_u) - start
            hi_j = jnp.minimum(hi, start + B) - start

            def grp_body(gi, c):
                b0, b1, b2, b3, pr = c
                j0 = gi * 16
                rv = rows_v[pl.ds(j0, 16)] - row_base
                vv = vals_v[pl.ds(j0, 16)]
                for lane in range(16):
                    jj = j0 + lane
                    ok = (jj >= lo_j) & (jj < hi_j)
                    r = jnp.where(ok, rv[lane], pr)
                    v = jnp.where(ok, vv[lane], 0.0)
                    keep = jnp.where(r == pr, 1.0, 0.0)
                    b0 = b0 * keep + v * g[jj, 0:16]
                    b1 = b1 * keep + v * g[jj, 16:32]
                    b2 = b2 * keep + v * g[jj, 32:48]
                    b3 = b3 * keep + v * g[jj, 48:64]
                    win[r, 0:16] = b0
                    win[r, 16:32] = b1
                    win[r, 32:48] = b2
                    win[r, 48:64] = b3
                    pr = r
                return (b0, b1, b2, b3, pr)

            return lax.fori_loop(lo_j // 16, (hi_j + 15) // 16, grp_body,
                                 (a0, a1, a2, a3, prev))

        lax.fori_loop(0, nblk, blk_body,
                      (zvec, zvec, zvec, zvec, jnp.int32(0)))

        pltpu.sync_copy(win, out_hbm.at[pl.ds(row_base, ROWS_PER_VW), :])


@jax.jit
def _sc_spmm(values, row32, col32, W, off):
    mesh = plsc.VectorSubcoreMesh(core_axis_name="c", subcore_axis_name="s")
    return pl.kernel(
        _sc_body,
        out_type=jax.ShapeDtypeStruct((BATCH, OUT), jnp.float32),
        mesh=mesh,
        compiler_params=pltpu.CompilerParams(use_tc_tiling_on_sc=False),
        scratch_types=[
            pltpu.VMEM((B,), jnp.int32),
            pltpu.VMEM((B, OUT), jnp.float32),
            pltpu.VMEM((ROWS_PER_VW, OUT), jnp.float32),
            pltpu.VMEM((B,), jnp.int32),
            pltpu.VMEM((B,), jnp.float32),
            pltpu.VMEM((16,), jnp.int32),
            pltpu.SemaphoreType.DMA,
        ],
    )(values, row32, col32, W, off)


def kernel(values, row_indices, col_indices, W):
    row32 = row_indices.astype(jnp.int32)
    col32 = col_indices.astype(jnp.int32)
    bounds = jnp.searchsorted(
        row32, (jnp.arange(NVW + 1, dtype=jnp.int32) * ROWS_PER_VW)
    ).astype(jnp.int32)
    off = jnp.concatenate(
        [bounds, jnp.full((15,), NNZ, dtype=jnp.int32)])
    return _sc_spmm(values, row32, col32, W, off)
